# Initial kernel scaffold; baseline (speedup 1.0000x reference)
import jax
import jax.numpy as jnp
from jax.experimental import pallas as pl


def kernel(x, edge_index, Wl1, bl1, Wr1, g1, b1, Wl2, bl2, Wr2, g2, b2, Wl3, bl3, Wr3):
    raise NotImplementedError("write your pallas kernel here")


# speedup vs baseline: 3.9593x; 3.9593x over previous
import jax
import jax.numpy as jnp
from jax import lax
from jax.experimental import pallas as pl
from jax.experimental.pallas import tpu as pltpu
from jax.experimental.pallas import tpu_sc as plsc

N = 10000
DIN = 128
DH = 128
DOUT = 47
EPS = 1e-5

N_PAD = 10240
DHALF = DH // 2
BM = 512
EB = 128
CB = 4
NC = 2
NS = 16
ROWS_PER_TILE = N_PAD // NS


def _make_agg(rb_per_worker: int, with_cnt: bool):
    mesh = plsc.VectorSubcoreMesh(core_axis_name="c", subcore_axis_name="s")
    out_type = [jax.ShapeDtypeStruct((NC, N_PAD, DHALF), jnp.float32)]
    scratch = [
        pltpu.VMEM((2 * CB, EB), jnp.int32),
        pltpu.VMEM((2 * CB, EB), jnp.int32),
        pltpu.VMEM((CB, EB, DHALF), jnp.float32),
        pltpu.VMEM_SHARED((N_PAD, DHALF), jnp.float32),
        pltpu.SemaphoreType.DMA,
    ]
    if with_cnt:
        out_type.append(jax.ShapeDtypeStruct((NC, N_PAD, 16), jnp.float32))
        scratch += [
            pltpu.VMEM((EB, 16), jnp.float32),
            pltpu.VMEM_SHARED((N_PAD, 16), jnp.float32),
        ]

    def body(y_hbm, src_hbm, dst_hbm, zeros_hbm, zeros16_hbm, ones_hbm,
             acc_out, *rest):
        if with_cnt:
            cnt_out, src_v, dst_v, rows_v, acc_sh, sem, ones_v, cnt_sh = rest
        else:
            src_v, dst_v, rows_v, acc_sh, sem = rest
        c = lax.axis_index("c")
        s = lax.axis_index("s")
        row0 = s * ROWS_PER_TILE

        pltpu.sync_copy(zeros_hbm, acc_sh.at[pl.ds(row0, ROWS_PER_TILE)])
        if with_cnt:
            pltpu.sync_copy(zeros16_hbm, cnt_sh.at[pl.ds(row0, ROWS_PER_TILE)])
            pltpu.sync_copy(ones_hbm, ones_v)
        plsc.subcore_barrier()

        @pl.loop(0, rb_per_worker // (2 * CB))
        def _(blk):
            base = s * rb_per_worker + blk * (2 * CB)
            pltpu.sync_copy(src_hbm.at[c].at[pl.ds(base, 2 * CB)], src_v)
            pltpu.sync_copy(dst_hbm.at[pl.ds(base, 2 * CB)], dst_v)
            for half in range(2):
                descs = [
                    pltpu.async_copy(
                        y_hbm.at[src_v.at[half * CB + j]], rows_v.at[j], sem)
                    for j in range(CB)
                ]
                for dsc in descs:
                    dsc.wait()
                for j in range(CB):
                    pltpu.sync_copy(rows_v.at[j],
                                    acc_sh.at[dst_v.at[half * CB + j]],
                                    add=True)
                    if with_cnt:
                        pltpu.sync_copy(ones_v,
                                        cnt_sh.at[dst_v.at[half * CB + j]],
                                        add=True)

        plsc.subcore_barrier()
        sl = pl.ds(row0, ROWS_PER_TILE)
        pltpu.sync_copy(acc_sh.at[sl], acc_out.at[c].at[sl])
        if with_cnt:
            pltpu.sync_copy(cnt_sh.at[sl], cnt_out.at[c].at[sl])

    return pl.kernel(body, out_type=tuple(out_type), mesh=mesh,
                     scratch_types=tuple(scratch),
                     compiler_params=pltpu.CompilerParams(
                         use_tc_tiling_on_sc=False))


def _mm_body(x_ref, w_ref, o_ref):
    o_ref[...] = jnp.dot(x_ref[...], w_ref[...],
                         preferred_element_type=jnp.float32)


def _matmul(x, w):
    n, k = x.shape
    m = w.shape[1]
    return pl.pallas_call(
        _mm_body,
        grid=(n // BM,),
        in_specs=[pl.BlockSpec((BM, k), lambda i: (i, 0)),
                  pl.BlockSpec((k, m), lambda i: (0, 0))],
        out_specs=pl.BlockSpec((BM, m), lambda i: (i, 0)),
        out_shape=jax.ShapeDtypeStruct((n, m), jnp.float32),
    )(x, w)


def _combine_body(acc_ref, cnt_ref, x_ref, wr_ref, wn_ref, bl_ref, g_ref,
                  beta_ref, h_ref, y_ref):
    cnt = cnt_ref[0, :, 0:1]
    inv = 1.0 / jnp.maximum(cnt, 1.0)
    agg = jnp.concatenate([acc_ref[0], acc_ref[1]], axis=1)
    h = agg * inv + bl_ref[...] + jnp.dot(
        x_ref[...], wr_ref[...], preferred_element_type=jnp.float32)
    scale = g_ref[...] * (1.0 / jnp.sqrt(1.0 + EPS))
    h = jnp.maximum(scale * h + beta_ref[...], 0.0)
    h_ref[...] = h
    y_ref[...] = jnp.dot(h, wn_ref[...], preferred_element_type=jnp.float32)


def _combine(acc, cnt, x, wrT, wnextT, bl, g, beta):
    return pl.pallas_call(
        _combine_body,
        grid=(N_PAD // BM,),
        in_specs=[
            pl.BlockSpec((NC, BM, DHALF), lambda i: (0, i, 0)),
            pl.BlockSpec((1, BM, 16), lambda i: (0, i, 0)),
            pl.BlockSpec((BM, DH), lambda i: (i, 0)),
            pl.BlockSpec((DH, DH), lambda i: (0, 0)),
            pl.BlockSpec((DH, DH), lambda i: (0, 0)),
            pl.BlockSpec((1, DH), lambda i: (0, 0)),
            pl.BlockSpec((1, DH), lambda i: (0, 0)),
            pl.BlockSpec((1, DH), lambda i: (0, 0)),
        ],
        out_specs=[pl.BlockSpec((BM, DH), lambda i: (i, 0)),
                   pl.BlockSpec((BM, DH), lambda i: (i, 0))],
        out_shape=[jax.ShapeDtypeStruct((N_PAD, DH), jnp.float32),
                   jax.ShapeDtypeStruct((N_PAD, DH), jnp.float32)],
    )(acc, cnt, x, wrT, wnextT, bl, g, beta)


def _final_body(acc_ref, cnt_ref, h_ref, wr_ref, bl_ref, o_ref):
    cnt = cnt_ref[0, :, 0:1]
    inv = 1.0 / jnp.maximum(cnt, 1.0)
    agg = jnp.concatenate([acc_ref[0], acc_ref[1]], axis=1)
    z = agg * inv + bl_ref[...] + jnp.dot(
        h_ref[...], wr_ref[...], preferred_element_type=jnp.float32)
    col = lax.broadcasted_iota(jnp.int32, z.shape, 1)
    valid = col < DOUT
    zm = jnp.where(valid, z, -jnp.inf)
    m = jnp.max(zm, axis=1, keepdims=True)
    e = jnp.where(valid, jnp.exp(z - m), 0.0)
    lse = jnp.log(jnp.sum(e, axis=1, keepdims=True))
    o_ref[...] = z - m - lse


def _final(acc, cnt, h, wrT, bl):
    return pl.pallas_call(
        _final_body,
        grid=(N_PAD // BM,),
        in_specs=[
            pl.BlockSpec((NC, BM, DHALF), lambda i: (0, i, 0)),
            pl.BlockSpec((1, BM, 16), lambda i: (0, i, 0)),
            pl.BlockSpec((BM, DH), lambda i: (i, 0)),
            pl.BlockSpec((DH, DH), lambda i: (0, 0)),
            pl.BlockSpec((1, DH), lambda i: (0, 0)),
        ],
        out_specs=pl.BlockSpec((BM, DH), lambda i: (i, 0)),
        out_shape=jax.ShapeDtypeStruct((N_PAD, DH), jnp.float32),
    )(acc, cnt, h, wrT, bl)


def _split_cols(y):
    return jnp.concatenate([y[:, :DHALF], y[:, DHALF:]], axis=0)


def kernel(x, edge_index, Wl1, bl1, Wr1, g1, b1, Wl2, bl2, Wr2, g2, b2,
           Wl3, bl3, Wr3):
    e = edge_index.shape[1]
    blk_edges = NS * EB * 2 * CB
    e_pad = ((e + blk_edges - 1) // blk_edges) * blk_edges
    rb_total = e_pad // EB
    rb_per_worker = rb_total // NS

    src1 = jnp.concatenate(
        [edge_index[0], jnp.zeros((e_pad - e,), jnp.int32)]).reshape(rb_total, EB)
    src = jnp.stack([src1, src1 + N_PAD])
    dst = jnp.concatenate(
        [edge_index[1], jnp.full((e_pad - e,), N, jnp.int32)]).reshape(rb_total, EB)

    x_p = jnp.concatenate([x, jnp.zeros((N_PAD - N, DIN), jnp.float32)])
    zeros_d = jnp.zeros((ROWS_PER_TILE, DHALF), jnp.float32)
    zeros16 = jnp.zeros((ROWS_PER_TILE, 16), jnp.float32)
    ones16 = jnp.ones((EB, 16), jnp.float32)

    wl3T = jnp.zeros((DH, DH), jnp.float32).at[:, :DOUT].set(Wl3.T)
    wr3T = jnp.zeros((DH, DH), jnp.float32).at[:, :DOUT].set(Wr3.T)
    bl3p = jnp.zeros((1, DH), jnp.float32).at[0, :DOUT].set(bl3)

    agg_cnt = _make_agg(rb_per_worker, True)
    agg = _make_agg(rb_per_worker, False)

    r2 = lambda v: v.reshape(1, DH)

    y1 = _matmul(x_p, Wl1.T)
    acc1, cnt = agg_cnt(_split_cols(y1), src, dst, zeros_d, zeros16, ones16)
    h1, y2 = _combine(acc1, cnt, x_p, Wr1.T, Wl2.T, r2(bl1), r2(g1), r2(b1))
    (acc2,) = agg(_split_cols(y2), src, dst, zeros_d, zeros16, ones16)
    h2, y3 = _combine(acc2, cnt, h1, Wr2.T, wl3T, r2(bl2), r2(g2), r2(b2))
    (acc3,) = agg(_split_cols(y3), src, dst, zeros_d, zeros16, ones16)
    o = _final(acc3, cnt, h2, wr3T, bl3p)
    return o[:N, :DOUT]


# speedup vs baseline: 4.3261x; 1.0926x over previous
import jax
import jax.numpy as jnp
from jax import lax
from jax.experimental import pallas as pl
from jax.experimental.pallas import tpu as pltpu
from jax.experimental.pallas import tpu_sc as plsc

N = 10000
DIN = 128
DH = 128
DOUT = 47
EPS = 1e-5

N_PAD = 10240
DHALF = DH // 2
BM = 512
EB = 128
CB = 4
NC = 2
NS = 16
ROWS_PER_TILE = N_PAD // NS


def _make_agg(rb_per_worker: int, with_cnt: bool):
    mesh = plsc.VectorSubcoreMesh(core_axis_name="c", subcore_axis_name="s")
    out_type = [jax.ShapeDtypeStruct((NC, N_PAD, DHALF), jnp.float32)]
    scratch = [
        pltpu.VMEM((2 * CB, EB), jnp.int32),
        pltpu.VMEM((2 * CB, EB), jnp.int32),
        pltpu.VMEM((2 * CB, EB), jnp.int32),
        pltpu.VMEM((2 * CB, EB), jnp.int32),
        pltpu.VMEM((CB, EB, DHALF), jnp.float32),
        pltpu.VMEM((CB, EB, DHALF), jnp.float32),
        pltpu.VMEM_SHARED((N_PAD, DHALF), jnp.float32),
        pltpu.SemaphoreType.DMA,
        pltpu.SemaphoreType.DMA,
        pltpu.SemaphoreType.DMA,
        pltpu.SemaphoreType.DMA,
        pltpu.SemaphoreType.DMA,
        pltpu.SemaphoreType.DMA,
    ]
    if with_cnt:
        out_type.append(jax.ShapeDtypeStruct((NC, N_PAD, 16), jnp.float32))
        scratch += [
            pltpu.VMEM((EB, 16), jnp.float32),
            pltpu.VMEM_SHARED((N_PAD, 16), jnp.float32),
        ]

    n_iters = rb_per_worker // (4 * CB)

    def body(y_hbm, src_hbm, dst_hbm, zeros_hbm, zeros16_hbm, ones_hbm,
             acc_out, *rest):
        if with_cnt:
            (cnt_out, srcP, dstP, srcQ, dstQ, bufA, bufB, acc_sh,
             semGA, semGB, semSA, semSB, semIP, semIQ, ones_v, cnt_sh) = rest
        else:
            (srcP, dstP, srcQ, dstQ, bufA, bufB, acc_sh,
             semGA, semGB, semSA, semSB, semIP, semIQ) = rest
        c = lax.axis_index("c")
        s = lax.axis_index("s")
        row0 = s * ROWS_PER_TILE
        tile_base = s * rb_per_worker

        pltpu.sync_copy(zeros_hbm, acc_sh.at[pl.ds(row0, ROWS_PER_TILE)])
        if with_cnt:
            pltpu.sync_copy(zeros16_hbm, cnt_sh.at[pl.ds(row0, ROWS_PER_TILE)])
            pltpu.sync_copy(ones_hbm, ones_v)
        plsc.subcore_barrier()

        def load_idx(sbuf, dbuf, base, sem):
            return [
                pltpu.async_copy(src_hbm.at[c].at[pl.ds(base, 2 * CB)],
                                 sbuf, sem),
                pltpu.async_copy(dst_hbm.at[pl.ds(base, 2 * CB)], dbuf, sem),
            ]

        def fire_gathers(sbuf, off, rowbuf, sem):
            return [
                pltpu.async_copy(y_hbm.at[sbuf.at[off + j]], rowbuf.at[j], sem)
                for j in range(CB)
            ]

        def fire_scatters(dbuf, off, rowbuf, sem):
            return [
                pltpu.async_copy(rowbuf.at[j], acc_sh.at[dbuf.at[off + j]],
                                 sem, add=True)
                for j in range(CB)
            ]

        def cnt_adds(dbuf, off, pred):
            if not with_cnt:
                return

            @pl.when(pred)
            def _():
                for j in range(CB):
                    pltpu.sync_copy(ones_v, cnt_sh.at[dbuf.at[off + j]],
                                    add=True)

        def drain(descs):
            for d in descs:
                d.wait()

        @pl.loop(0, n_iters)
        def _(m):
            pred = (c == 0) == (m < n_iters // 2)
            base = tile_base + m * (4 * CB)
            iP = load_idx(srcP, dstP, base, semIP)
            drain(iP)
            gA = fire_gathers(srcP, 0, bufA, semGA)
            gB = fire_gathers(srcP, CB, bufB, semGB)
            iQ = load_idx(srcQ, dstQ, base + 2 * CB, semIQ)
            drain(gA)
            sA = fire_scatters(dstP, 0, bufA, semSA)
            cnt_adds(dstP, 0, pred)
            drain(gB)
            sB = fire_scatters(dstP, CB, bufB, semSB)
            cnt_adds(dstP, CB, pred)
            drain(sA)
            drain(iQ)
            gA2 = fire_gathers(srcQ, 0, bufA, semGA)
            drain(sB)
            gB2 = fire_gathers(srcQ, CB, bufB, semGB)
            drain(gA2)
            sA2 = fire_scatters(dstQ, 0, bufA, semSA)
            cnt_adds(dstQ, 0, pred)
            drain(gB2)
            sB2 = fire_scatters(dstQ, CB, bufB, semSB)
            cnt_adds(dstQ, CB, pred)
            drain(sA2)
            drain(sB2)

        plsc.subcore_barrier()
        sl = pl.ds(row0, ROWS_PER_TILE)
        pltpu.sync_copy(acc_sh.at[sl], acc_out.at[c].at[sl])
        if with_cnt:
            pltpu.sync_copy(cnt_sh.at[sl], cnt_out.at[c].at[sl])

    return pl.kernel(body, out_type=tuple(out_type), mesh=mesh,
                     scratch_types=tuple(scratch),
                     compiler_params=pltpu.CompilerParams(
                         use_tc_tiling_on_sc=False))


def _mm_body(x_ref, w_ref, o_ref):
    o_ref[...] = jnp.dot(x_ref[...], w_ref[...],
                         preferred_element_type=jnp.float32)


def _matmul(x, w):
    n, k = x.shape
    m = w.shape[1]
    return pl.pallas_call(
        _mm_body,
        grid=(n // BM,),
        in_specs=[pl.BlockSpec((BM, k), lambda i: (i, 0)),
                  pl.BlockSpec((k, m), lambda i: (0, 0))],
        out_specs=pl.BlockSpec((BM, m), lambda i: (i, 0)),
        out_shape=jax.ShapeDtypeStruct((n, m), jnp.float32),
    )(x, w)


def _combine_body(acc_ref, cnt_ref, x_ref, wr_ref, wn_ref, bl_ref, g_ref,
                  beta_ref, h_ref, y_ref):
    cnt = cnt_ref[0, :, 0:1] + cnt_ref[1, :, 0:1]
    inv = 1.0 / jnp.maximum(cnt, 1.0)
    agg = jnp.concatenate([acc_ref[0], acc_ref[1]], axis=1)
    h = agg * inv + bl_ref[...] + jnp.dot(
        x_ref[...], wr_ref[...], preferred_element_type=jnp.float32)
    scale = g_ref[...] * (1.0 / jnp.sqrt(1.0 + EPS))
    h = jnp.maximum(scale * h + beta_ref[...], 0.0)
    h_ref[...] = h
    y_ref[...] = jnp.dot(h, wn_ref[...], preferred_element_type=jnp.float32)


def _combine(acc, cnt, x, wrT, wnextT, bl, g, beta):
    return pl.pallas_call(
        _combine_body,
        grid=(N_PAD // BM,),
        in_specs=[
            pl.BlockSpec((NC, BM, DHALF), lambda i: (0, i, 0)),
            pl.BlockSpec((NC, BM, 16), lambda i: (0, i, 0)),
            pl.BlockSpec((BM, DH), lambda i: (i, 0)),
            pl.BlockSpec((DH, DH), lambda i: (0, 0)),
            pl.BlockSpec((DH, DH), lambda i: (0, 0)),
            pl.BlockSpec((1, DH), lambda i: (0, 0)),
            pl.BlockSpec((1, DH), lambda i: (0, 0)),
            pl.BlockSpec((1, DH), lambda i: (0, 0)),
        ],
        out_specs=[pl.BlockSpec((BM, DH), lambda i: (i, 0)),
                   pl.BlockSpec((BM, DH), lambda i: (i, 0))],
        out_shape=[jax.ShapeDtypeStruct((N_PAD, DH), jnp.float32),
                   jax.ShapeDtypeStruct((N_PAD, DH), jnp.float32)],
    )(acc, cnt, x, wrT, wnextT, bl, g, beta)


def _final_body(acc_ref, cnt_ref, h_ref, wr_ref, bl_ref, o_ref):
    cnt = cnt_ref[0, :, 0:1] + cnt_ref[1, :, 0:1]
    inv = 1.0 / jnp.maximum(cnt, 1.0)
    agg = jnp.concatenate([acc_ref[0], acc_ref[1]], axis=1)
    z = agg * inv + bl_ref[...] + jnp.dot(
        h_ref[...], wr_ref[...], preferred_element_type=jnp.float32)
    col = lax.broadcasted_iota(jnp.int32, z.shape, 1)
    valid = col < DOUT
    zm = jnp.where(valid, z, -jnp.inf)
    m = jnp.max(zm, axis=1, keepdims=True)
    e = jnp.where(valid, jnp.exp(z - m), 0.0)
    lse = jnp.log(jnp.sum(e, axis=1, keepdims=True))
    o_ref[...] = z - m - lse


def _final(acc, cnt, h, wrT, bl):
    return pl.pallas_call(
        _final_body,
        grid=(N_PAD // BM,),
        in_specs=[
            pl.BlockSpec((NC, BM, DHALF), lambda i: (0, i, 0)),
            pl.BlockSpec((NC, BM, 16), lambda i: (0, i, 0)),
            pl.BlockSpec((BM, DH), lambda i: (i, 0)),
            pl.BlockSpec((DH, DH), lambda i: (0, 0)),
            pl.BlockSpec((1, DH), lambda i: (0, 0)),
        ],
        out_specs=pl.BlockSpec((BM, DH), lambda i: (i, 0)),
        out_shape=jax.ShapeDtypeStruct((N_PAD, DH), jnp.float32),
    )(acc, cnt, h, wrT, bl)


def _split_cols(y):
    return jnp.concatenate([y[:, :DHALF], y[:, DHALF:]], axis=0)


def kernel(x, edge_index, Wl1, bl1, Wr1, g1, b1, Wl2, bl2, Wr2, g2, b2,
           Wl3, bl3, Wr3):
    e = edge_index.shape[1]
    blk_edges = NS * EB * 4 * CB
    e_pad = ((e + blk_edges - 1) // blk_edges) * blk_edges
    rb_total = e_pad // EB
    rb_per_worker = rb_total // NS

    src1 = jnp.concatenate(
        [edge_index[0], jnp.zeros((e_pad - e,), jnp.int32)]).reshape(rb_total, EB)
    src = jnp.stack([src1, src1 + N_PAD])
    dst = jnp.concatenate(
        [edge_index[1], jnp.full((e_pad - e,), N, jnp.int32)]).reshape(rb_total, EB)

    x_p = jnp.concatenate([x, jnp.zeros((N_PAD - N, DIN), jnp.float32)])
    zeros_d = jnp.zeros((ROWS_PER_TILE, DHALF), jnp.float32)
    zeros16 = jnp.zeros((ROWS_PER_TILE, 16), jnp.float32)
    ones16 = jnp.ones((EB, 16), jnp.float32)

    wl3T = jnp.zeros((DH, DH), jnp.float32).at[:, :DOUT].set(Wl3.T)
    wr3T = jnp.zeros((DH, DH), jnp.float32).at[:, :DOUT].set(Wr3.T)
    bl3p = jnp.zeros((1, DH), jnp.float32).at[0, :DOUT].set(bl3)

    agg_cnt = _make_agg(rb_per_worker, True)
    agg = _make_agg(rb_per_worker, False)

    r2 = lambda v: v.reshape(1, DH)

    y1 = _matmul(x_p, Wl1.T)
    acc1, cnt = agg_cnt(_split_cols(y1), src, dst, zeros_d, zeros16, ones16)
    h1, y2 = _combine(acc1, cnt, x_p, Wr1.T, Wl2.T, r2(bl1), r2(g1), r2(b1))
    (acc2,) = agg(_split_cols(y2), src, dst, zeros_d, zeros16, ones16)
    h2, y3 = _combine(acc2, cnt, h1, Wr2.T, wl3T, r2(bl2), r2(g2), r2(b2))
    (acc3,) = agg(_split_cols(y3), src, dst, zeros_d, zeros16, ones16)
    o = _final(acc3, cnt, h2, wr3T, bl3p)
    return o[:N, :DOUT]
